# baseline (speedup 1.0000x reference)
import functools

import jax
import jax.numpy as jnp
from jax import lax
from jax.experimental import pallas as pl
from jax.experimental.pallas import tpu as pltpu
from jax.experimental.pallas import tpu_sc as plsc

_N, _C, _K, _OUT = 10000, 128, 32, 128
_NP = 10240
_NG = 2
_NH = 16
_NPG = _NP // _NG
_CH = _C // _NH
_PC = _CH // 2
_CB = 256
_NCHUNK = _NPG // _CB
_GRPS = _CB // 16
_NBUF = 2
_LANES = 16
_NEG_INF = float("-inf")


def _maxdiff_body(xs_hbm, idx_hbm, out_hbm, slab, idx_v, out_v, xsh,
                  sem_i, sem_o):
    wid = lax.axis_index("s") * 2 + lax.axis_index("c")
    g = wid // _NH
    h = wid % _NH
    sid = lax.axis_index("s")

    pltpu.sync_copy(xs_hbm.at[pl.ds(sid * _PC, _PC)],
                    xsh.at[pl.ds(sid * _PC, _PC)])
    plsc.subcore_barrier()
    pltpu.sync_copy(xsh.at[pl.ds(h * _PC, _PC)], slab)

    def fetch(buf, ci):
        pltpu.async_copy(idx_hbm.at[g, ci], idx_v.at[buf], sem_i.at[buf])

    def drain_idx(buf):
        pltpu.make_async_copy(idx_hbm.at[0, 0], idx_v.at[buf],
                              sem_i.at[buf]).wait()

    for b in range(_NBUF):
        fetch(b, b)

    def seg_body(s, carry):
        for b in range(_NBUF):
            ci = s * _NBUF + b
            drain_idx(b)
            @pl.when(s > 0)
            def _(b=b):
                pltpu.make_async_copy(out_v.at[b], out_hbm.at[0, 0, 0],
                                      sem_o.at[b]).wait()

            def grp_body(grp, carry, b=b):
                accs = [jnp.full((_LANES,), _NEG_INF, dtype=jnp.float32)
                        for _ in range(_CH)]
                col = grp * _LANES
                for k in range(_K):
                    w = idx_v[b, k, pl.ds(col, _LANES)]
                    jv = jnp.bitwise_and(w, 0xFFFF)
                    iv = lax.shift_right_logical(w, 16)
                    for pc in range(_PC):
                        row = jnp.full((_LANES,), pc, dtype=jnp.int32)
                        wj = plsc.load_gather(slab, [row, jv])
                        wi = plsc.load_gather(slab, [row, iv])
                        jlo = plsc.bitcast(lax.shift_left(wj, 16),
                                           jnp.float32)
                        ilo = plsc.bitcast(lax.shift_left(wi, 16),
                                           jnp.float32)
                        jhi = plsc.bitcast(
                            jnp.bitwise_and(wj, -65536), jnp.float32)
                        ihi = plsc.bitcast(
                            jnp.bitwise_and(wi, -65536), jnp.float32)
                        accs[2 * pc] = jnp.maximum(accs[2 * pc], jlo - ilo)
                        accs[2 * pc + 1] = jnp.maximum(accs[2 * pc + 1],
                                                       jhi - ihi)
                for q in range(_CH):
                    out_v[b, q, pl.ds(col, _LANES)] = accs[q]
                return carry

            lax.fori_loop(0, _GRPS, grp_body, 0)
            pltpu.async_copy(out_v.at[b], out_hbm.at[g, ci, h], sem_o.at[b])
            @pl.when(ci + _NBUF < _NCHUNK)
            def _(b=b, ci=ci):
                fetch(b, ci + _NBUF)
        return carry

    lax.fori_loop(0, _NCHUNK // _NBUF, seg_body, 0)
    for b in range(_NBUF):
        pltpu.make_async_copy(out_v.at[b], out_hbm.at[0, 0, 0],
                              sem_o.at[b]).wait()


_maxdiff_kernel = functools.partial(
    pl.kernel,
    mesh=plsc.VectorSubcoreMesh(core_axis_name="c", subcore_axis_name="s"),
    compiler_params=pltpu.CompilerParams(needs_layout_passes=False),
    out_type=jax.ShapeDtypeStruct((_NG, _NCHUNK, _NH, _CH, _CB),
                                  jnp.float32),
    scratch_types=[
        pltpu.VMEM((_PC, _NP), jnp.int32),
        pltpu.VMEM((_NBUF, _K, _CB), jnp.int32),
        pltpu.VMEM((_NBUF, _CH, _CB), jnp.float32),
        pltpu.VMEM_SHARED((_NH * _PC, _NP), jnp.int32),
        pltpu.SemaphoreType.DMA((_NBUF,)),
        pltpu.SemaphoreType.DMA((_NBUF,)),
    ],
)(_maxdiff_body)


_TN = 1024


def _mlp_body(x_ref, md_ref, we_ref, wo_ref, b_ref, o_ref):
    acc = jnp.dot(x_ref[...], we_ref[...], preferred_element_type=jnp.float32)
    acc = acc + jnp.dot(md_ref[...], wo_ref[...],
                        preferred_element_type=jnp.float32)
    o_ref[...] = jnp.maximum(acc + b_ref[...], 0.0)


def kernel(x, edge_index, W, bparam):
    x2 = x[0]

    xu = jax.lax.bitcast_convert_type(
        x2.astype(jnp.bfloat16), jnp.uint16).astype(jnp.uint32)
    xw = (xu[:, 0::2] | (xu[:, 1::2] << 16)).astype(jnp.int32)
    xs = jnp.pad(xw, ((0, _NP - _N), (0, 0))).T

    idx = edge_index[:, 0].astype(jnp.int32)
    idx = jnp.pad(idx, ((0, 0), (0, _NP - _N), (0, 0)))
    pji = idx[0] | (idx[1] << 16)
    pji = pji.reshape(_NG, _NCHUNK, _CB, _K).transpose(0, 1, 3, 2)

    mdx = _maxdiff_kernel(xs, pji)
    maxdiff = mdx.transpose(0, 1, 4, 2, 3).reshape(_NP, _C)

    xp = jnp.pad(x2, ((0, _NP - _N), (0, 0)))
    we_t = W[:, 0::2].T
    wo_t = W[:, 1::2].T
    b2 = bparam.reshape(1, _OUT)

    out = pl.pallas_call(
        _mlp_body,
        grid=(_NP // _TN,),
        in_specs=[
            pl.BlockSpec((_TN, _C), lambda i: (i, 0)),
            pl.BlockSpec((_TN, _C), lambda i: (i, 0)),
            pl.BlockSpec((_C, _OUT), lambda i: (0, 0)),
            pl.BlockSpec((_C, _OUT), lambda i: (0, 0)),
            pl.BlockSpec((1, _OUT), lambda i: (0, 0)),
        ],
        out_specs=pl.BlockSpec((_TN, _OUT), lambda i: (i, 0)),
        out_shape=jax.ShapeDtypeStruct((_NP, _OUT), jnp.float32),
    )(xp, maxdiff, we_t, wo_t, b2)

    return out[:_N][None]


# speedup vs baseline: 1.0220x; 1.0220x over previous
import functools

import jax
import jax.numpy as jnp
from jax import lax
from jax.experimental import pallas as pl
from jax.experimental.pallas import tpu as pltpu
from jax.experimental.pallas import tpu_sc as plsc

_N, _C, _K, _OUT = 10000, 128, 32, 128
_NP = 10240
_NG = 2
_NH = 16
_NPG = _NP // _NG
_CH = _C // _NH
_PC = _CH // 2
_CB = 256
_NCHUNK = _NPG // _CB
_GRPS = _CB // 16
_NBUF = 2
_LANES = 16
_NEG_INF = float("-inf")


def _maxdiff_body(xs_hbm, idx_hbm, out_hbm, slab, idx_v, out_v, xsh,
                  sem_i, sem_o):
    wid = lax.axis_index("s") * 2 + lax.axis_index("c")
    g = wid // _NH
    h = wid % _NH
    sid = lax.axis_index("s")

    pltpu.sync_copy(xs_hbm.at[pl.ds(sid * _PC, _PC)],
                    xsh.at[pl.ds(sid * _PC, _PC)])
    plsc.subcore_barrier()
    pltpu.sync_copy(xsh.at[pl.ds(h * _PC, _PC)], slab)

    def fetch(buf, ci):
        pltpu.async_copy(idx_hbm.at[g, ci], idx_v.at[buf], sem_i.at[buf])

    def drain_idx(buf):
        pltpu.make_async_copy(idx_hbm.at[0, 0], idx_v.at[buf],
                              sem_i.at[buf]).wait()

    for b in range(_NBUF):
        fetch(b, b)

    def seg_body(s, carry):
        for b in range(_NBUF):
            ci = s * _NBUF + b
            drain_idx(b)
            @pl.when(s > 0)
            def _(b=b):
                pltpu.make_async_copy(out_v.at[b], out_hbm.at[0, 0, 0],
                                      sem_o.at[b]).wait()

            def grp_body(grp, carry, b=b):
                accs = [jnp.full((_LANES,), _NEG_INF, dtype=jnp.float32)
                        for _ in range(_CH)]
                col = grp * _LANES
                for k in range(_K):
                    w = idx_v[b, k, pl.ds(col, _LANES)]
                    jv = jnp.bitwise_and(w, 0xFFFF)
                    iv = lax.shift_right_logical(w, 16)
                    for pc in range(_PC):
                        row = jnp.full((_LANES,), pc, dtype=jnp.int32)
                        wj = plsc.load_gather(slab, [row, jv])
                        wi = plsc.load_gather(slab, [row, iv])
                        jlo = plsc.bitcast(lax.shift_left(wj, 16),
                                           jnp.float32)
                        ilo = plsc.bitcast(lax.shift_left(wi, 16),
                                           jnp.float32)
                        jhi = plsc.bitcast(
                            jnp.bitwise_and(wj, -65536), jnp.float32)
                        ihi = plsc.bitcast(
                            jnp.bitwise_and(wi, -65536), jnp.float32)
                        accs[2 * pc] = jnp.maximum(accs[2 * pc], jlo - ilo)
                        accs[2 * pc + 1] = jnp.maximum(accs[2 * pc + 1],
                                                       jhi - ihi)
                for q in range(_CH):
                    out_v[b, q, pl.ds(col, _LANES)] = accs[q]
                return carry

            lax.fori_loop(0, _GRPS, grp_body, 0)
            pltpu.async_copy(out_v.at[b], out_hbm.at[g, ci, h], sem_o.at[b])
            @pl.when(ci + _NBUF < _NCHUNK)
            def _(b=b, ci=ci):
                fetch(b, ci + _NBUF)
        return carry

    lax.fori_loop(0, _NCHUNK // _NBUF, seg_body, 0)
    for b in range(_NBUF):
        pltpu.make_async_copy(out_v.at[b], out_hbm.at[0, 0, 0],
                              sem_o.at[b]).wait()


_maxdiff_kernel = functools.partial(
    pl.kernel,
    mesh=plsc.VectorSubcoreMesh(core_axis_name="c", subcore_axis_name="s"),
    compiler_params=pltpu.CompilerParams(needs_layout_passes=False),
    out_type=jax.ShapeDtypeStruct((_NG, _NCHUNK, _NH, _CH, _CB),
                                  jnp.float32),
    scratch_types=[
        pltpu.VMEM((_PC, _NP), jnp.int32),
        pltpu.VMEM((_NBUF, _K, _CB), jnp.int32),
        pltpu.VMEM((_NBUF, _CH, _CB), jnp.float32),
        pltpu.VMEM_SHARED((_NH * _PC, _NP), jnp.int32),
        pltpu.SemaphoreType.DMA((_NBUF,)),
        pltpu.SemaphoreType.DMA((_NBUF,)),
    ],
)(_maxdiff_body)


_TN = _CB


def _mlp_body(x_ref, md_ref, we_ref, wo_ref, b_ref, o_ref):
    acc = jnp.dot(x_ref[...], we_ref[...], preferred_element_type=jnp.float32)
    md_t = md_ref[...].reshape(_C, _TN)
    acc = acc + lax.dot_general(md_t, wo_ref[...], (((0,), (0,)), ((), ())),
                                preferred_element_type=jnp.float32)
    o_ref[...] = jnp.maximum(acc + b_ref[...], 0.0)


def kernel(x, edge_index, W, bparam):
    x2 = x[0]

    xu = jax.lax.bitcast_convert_type(
        x2.astype(jnp.bfloat16), jnp.uint16).astype(jnp.uint32)
    xw = (xu[:, 0::2] | (xu[:, 1::2] << 16)).astype(jnp.int32)
    xs = jnp.pad(xw, ((0, _NP - _N), (0, 0))).T

    idx = edge_index[:, 0].astype(jnp.int32)
    idx = jnp.pad(idx, ((0, 0), (0, _NP - _N), (0, 0)))
    pji = idx[0] | (idx[1] << 16)
    pji = pji.reshape(_NG, _NCHUNK, _CB, _K).transpose(0, 1, 3, 2)

    mdx = _maxdiff_kernel(xs, pji)

    we_t = W[:, 0::2].T
    wo_t = W[:, 1::2].T
    b2 = bparam.reshape(1, _OUT)

    nblk = _NCHUNK
    out = pl.pallas_call(
        _mlp_body,
        grid=(pl.cdiv(_N, _TN),),
        in_specs=[
            pl.BlockSpec((_TN, _C), lambda i: (i, 0)),
            pl.BlockSpec((1, 1, _NH, _CH, _CB),
                         lambda i: (i // nblk, i % nblk, 0, 0, 0)),
            pl.BlockSpec((_C, _OUT), lambda i: (0, 0)),
            pl.BlockSpec((_C, _OUT), lambda i: (0, 0)),
            pl.BlockSpec((1, _OUT), lambda i: (0, 0)),
        ],
        out_specs=pl.BlockSpec((_TN, _OUT), lambda i: (i, 0)),
        out_shape=jax.ShapeDtypeStruct((_N, _OUT), jnp.float32),
    )(x2, mdx, we_t, wo_t, b2)

    return out[None]
